# baseline (speedup 1.0000x reference)
import jax
import jax.numpy as jnp
from jax.experimental import pallas as pl

N_NODES = 100000
N_EDGES = 1600000
NUM_GRAPHS = 200
N = 20
T = 10
NUM_TIMEPTS = 50
H = 32

NBLK = 2000
EBLK = 16000


def _uv_block(x_ref, wu_ref, wv_ref, out_ref):
    x = x_ref[...]
    u = jax.lax.dot_general(x, wu_ref[...], (((1,), (0,)), ((), ())),
                            preferred_element_type=jnp.float32)
    v = jax.lax.dot_general(x, wv_ref[...], (((1,), (0,)), ((), ())),
                            preferred_element_type=jnp.float32)
    out_ref[...] = jnp.concatenate([u, v], axis=1)


def _node_uv(x, Wu, Wv):
    F = x.shape[1]
    return pl.pallas_call(
        _uv_block,
        grid=(N_NODES // NBLK,),
        in_specs=[
            pl.BlockSpec((NBLK, F), lambda i: (i, 0)),
            pl.BlockSpec((F, H), lambda i: (0, 0)),
            pl.BlockSpec((F, H), lambda i: (0, 0)),
        ],
        out_specs=pl.BlockSpec((NBLK, 2 * H), lambda i: (i, 0)),
        out_shape=jax.ShapeDtypeStruct((N_NODES, 2 * H), jnp.float32),
    )(x, Wu, Wv)


def _msg_block(gu_ref, gv_ref, b1_ref, w2_ref, b2_ref, out_ref):
    t = jax.nn.relu(gu_ref[...] - gv_ref[...] + b1_ref[...])
    out_ref[...] = jax.lax.dot_general(
        t, w2_ref[...], (((1,), (0,)), ((), ())),
        preferred_element_type=jnp.float32) + b2_ref[...]


def _edge_msg(gu, gv, b1, W2, b2):
    E = gu.shape[0]
    return pl.pallas_call(
        _msg_block,
        grid=(E // EBLK,),
        in_specs=[
            pl.BlockSpec((EBLK, H), lambda i: (i, 0)),
            pl.BlockSpec((EBLK, H), lambda i: (i, 0)),
            pl.BlockSpec((1, H), lambda i: (0, 0)),
            pl.BlockSpec((H, H), lambda i: (0, 0)),
            pl.BlockSpec((1, H), lambda i: (0, 0)),
        ],
        out_specs=pl.BlockSpec((EBLK, H), lambda i: (i, 0)),
        out_shape=jax.ShapeDtypeStruct((E, H), jnp.float32),
    )(gu, gv, b1.reshape(1, H), W2, b2.reshape(1, H))


def _point_layer(h, pos_pad, src, dst, W1, b1, W2, b2, hw):
    Wh, Wp = W1[:hw], W1[hw:]
    F = pos_pad.shape[1] if hw == 3 else 40
    if hw == 3:
        x = pos_pad
        Wu = jnp.zeros((F, H), jnp.float32).at[:3].set(Wh + Wp)
        Wv = jnp.zeros((F, H), jnp.float32).at[:3].set(Wp)
    else:
        x = jnp.concatenate(
            [h, pos_pad[:, :3], jnp.zeros((N_NODES, F - hw - 3), jnp.float32)],
            axis=1)
        Wu = jnp.zeros((F, H), jnp.float32).at[:hw].set(Wh).at[hw:hw + 3].set(Wp)
        Wv = jnp.zeros((F, H), jnp.float32).at[hw:hw + 3].set(Wp)
    uv = _node_uv(x, Wu, Wv)
    u, v = uv[:, :H], uv[:, H:]
    m = _edge_msg(u[src], v[dst], b1, W2, b2)
    agg = jax.ops.segment_max(m, dst, num_segments=N_NODES)
    return jnp.where(jnp.isneginf(agg), 0.0, agg)


def kernel(pos, edge_index, batch, pts_tid, pts_msk, pts_aux,
           W1a, b1a, W2a, b2a, W1b, b1b, W2b, b2b):
    src, dst = edge_index[0], edge_index[1]
    pos_pad = jnp.pad(pos, ((0, 0), (0, 5)))
    h = jax.nn.relu(_point_layer(pos, pos_pad, src, dst, W1a, b1a, W2a, b2a, 3))
    h = jax.nn.relu(_point_layer(h, pos_pad, src, dst, W1b, b1b, W2b, b2b, H))
    enc_g = jax.ops.segment_max(h, batch, num_segments=NUM_GRAPHS)
    enc_g = jnp.where(jnp.isneginf(enc_g), 0.0, enc_g)
    enc = enc_g.reshape(N, T, H)
    parts_inp_obs = jnp.zeros((N, NUM_TIMEPTS, H), jnp.float32).at[:, :T].set(enc)
    parts_inp_msk = jnp.zeros((N, NUM_TIMEPTS, H), jnp.float32).at[:, :T].set(1.0)
    parts_inp_tps = jnp.zeros((N, NUM_TIMEPTS), jnp.float32).at[:, :T].set(
        pts_tid.astype(jnp.float32) / NUM_TIMEPTS)
    evd_obs = jnp.zeros((N, NUM_TIMEPTS, H), jnp.float32).at[
        jnp.arange(N)[:, None], pts_tid, :].set(enc)
    evd_msk = jnp.broadcast_to(pts_msk, (N, NUM_TIMEPTS, H))
    return (parts_inp_obs, parts_inp_msk, parts_inp_tps, evd_obs, evd_msk, pts_aux)


# speedup vs baseline: 1.8784x; 1.8784x over previous
import functools
import jax
import jax.numpy as jnp
from jax import lax
from jax.experimental import pallas as pl
from jax.experimental.pallas import tpu as pltpu
from jax.experimental.pallas import tpu_sc as plsc

N_NODES = 100000
N_EDGES = 1600000
NUM_GRAPHS = 200
N = 20
T = 10
NUM_TIMEPTS = 50
H = 32

NBLK = 2000
EBLK = 16384

NW = 32
GCHUNK = 128
MACRO = 512
CPW = 98
E_PAD = NW * MACRO * CPW


def _uv_block(x_ref, wu_ref, wv_ref, u_ref, v_ref):
    x = x_ref[...]
    u_ref[...] = jax.lax.dot_general(x, wu_ref[...], (((1,), (0,)), ((), ())),
                                     preferred_element_type=jnp.float32)
    v_ref[...] = jax.lax.dot_general(x, wv_ref[...], (((1,), (0,)), ((), ())),
                                     preferred_element_type=jnp.float32)


def _node_uv(x, Wu, Wv):
    F = x.shape[1]
    return pl.pallas_call(
        _uv_block,
        grid=(N_NODES // NBLK,),
        in_specs=[
            pl.BlockSpec((NBLK, F), lambda i: (i, 0)),
            pl.BlockSpec((F, H), lambda i: (0, 0)),
            pl.BlockSpec((F, H), lambda i: (0, 0)),
        ],
        out_specs=[pl.BlockSpec((NBLK, H), lambda i: (i, 0)),
                   pl.BlockSpec((NBLK, H), lambda i: (i, 0))],
        out_shape=[jax.ShapeDtypeStruct((N_NODES, H), jnp.float32),
                   jax.ShapeDtypeStruct((N_NODES, H), jnp.float32)],
    )(x, Wu, Wv)


def _gather_uv(u, v, src_pad, dst_pad):
    mesh = plsc.VectorSubcoreMesh(core_axis_name="c", subcore_axis_name="s")

    @functools.partial(
        pl.kernel, mesh=mesh,
        compiler_params=pltpu.CompilerParams(use_tc_tiling_on_sc=False),
        out_type=[jax.ShapeDtypeStruct((E_PAD, H), jnp.float32),
                  jax.ShapeDtypeStruct((E_PAD, H), jnp.float32)],
        scratch_types=[
            pltpu.VMEM((MACRO,), jnp.int32),
            pltpu.VMEM((MACRO,), jnp.int32),
            pltpu.VMEM((MACRO, H), jnp.float32),
            pltpu.VMEM((MACRO, H), jnp.float32),
            pltpu.SemaphoreType.DMA,
        ],
    )
    def k(u_hbm, v_hbm, src_hbm, dst_hbm, gu_hbm, gv_hbm,
          idxs, idxd, bufu, bufv, sem):
        wid = lax.axis_index("s") * 2 + lax.axis_index("c")
        base = wid * (MACRO * CPW)

        def body(c, carry):
            off = base + c * MACRO
            pltpu.sync_copy(src_hbm.at[pl.ds(off, MACRO)], idxs)
            pltpu.sync_copy(dst_hbm.at[pl.ds(off, MACRO)], idxd)
            cps = []
            for g in range(MACRO // GCHUNK):
                sl = pl.ds(g * GCHUNK, GCHUNK)
                cps.append(pltpu.async_copy(u_hbm.at[idxs.at[sl]],
                                            bufu.at[sl], sem))
                cps.append(pltpu.async_copy(v_hbm.at[idxd.at[sl]],
                                            bufv.at[sl], sem))
            for cp in cps:
                cp.wait()
            pltpu.sync_copy(bufu, gu_hbm.at[pl.ds(off, MACRO)])
            pltpu.sync_copy(bufv, gv_hbm.at[pl.ds(off, MACRO)])
            return carry

        lax.fori_loop(0, CPW, body, 0)

    return k(u, v, src_pad, dst_pad)


def _msg_block(gu_ref, gv_ref, b1_ref, w2_ref, b2_ref, out_ref):
    t = jax.nn.relu(gu_ref[...] - gv_ref[...] + b1_ref[...])
    out_ref[...] = jax.lax.dot_general(
        t, w2_ref[...], (((1,), (0,)), ((), ())),
        preferred_element_type=jnp.float32) + b2_ref[...]


def _edge_msg(gu, gv, b1, W2, b2):
    E = gu.shape[0]
    return pl.pallas_call(
        _msg_block,
        grid=(E // EBLK,),
        in_specs=[
            pl.BlockSpec((EBLK, H), lambda i: (i, 0)),
            pl.BlockSpec((EBLK, H), lambda i: (i, 0)),
            pl.BlockSpec((1, H), lambda i: (0, 0)),
            pl.BlockSpec((H, H), lambda i: (0, 0)),
            pl.BlockSpec((1, H), lambda i: (0, 0)),
        ],
        out_specs=pl.BlockSpec((EBLK, H), lambda i: (i, 0)),
        out_shape=jax.ShapeDtypeStruct((E, H), jnp.float32),
    )(gu, gv, b1.reshape(1, H), W2, b2.reshape(1, H))


def _point_layer(h, pos, src_pad, dst_pad, dst_seg, W1, b1, W2, b2, hw):
    Wh, Wp = W1[:hw], W1[hw:]
    F = 8 if hw == 3 else 40
    if hw == 3:
        x = jnp.pad(pos, ((0, 0), (0, F - 3)))
        Wu = jnp.zeros((F, H), jnp.float32).at[:3].set(Wh + Wp)
        Wv = jnp.zeros((F, H), jnp.float32).at[:3].set(Wp)
    else:
        x = jnp.concatenate(
            [h, pos, jnp.zeros((N_NODES, F - hw - 3), jnp.float32)], axis=1)
        Wu = jnp.zeros((F, H), jnp.float32).at[:hw].set(Wh).at[hw:hw + 3].set(Wp)
        Wv = jnp.zeros((F, H), jnp.float32).at[hw:hw + 3].set(Wp)
    u, v = _node_uv(x, Wu, Wv)
    gu, gv = _gather_uv(u, v, src_pad, dst_pad)
    m = _edge_msg(gu, gv, b1, W2, b2)
    agg = jax.ops.segment_max(m, dst_seg, num_segments=N_NODES + 1)[:N_NODES]
    return jnp.where(jnp.isneginf(agg), 0.0, agg)


def kernel(pos, edge_index, batch, pts_tid, pts_msk, pts_aux,
           W1a, b1a, W2a, b2a, W1b, b1b, W2b, b2b):
    src, dst = edge_index[0], edge_index[1]
    pad = E_PAD - N_EDGES
    src_pad = jnp.concatenate([src, jnp.zeros((pad,), src.dtype)])
    dst_pad = jnp.concatenate([dst, jnp.zeros((pad,), dst.dtype)])
    dst_seg = jnp.concatenate(
        [dst, jnp.full((pad,), N_NODES, dst.dtype)])
    h = jax.nn.relu(_point_layer(
        pos, pos, src_pad, dst_pad, dst_seg, W1a, b1a, W2a, b2a, 3))
    h = jax.nn.relu(_point_layer(
        h, pos, src_pad, dst_pad, dst_seg, W1b, b1b, W2b, b2b, H))
    enc_g = jax.ops.segment_max(h, batch, num_segments=NUM_GRAPHS)
    enc_g = jnp.where(jnp.isneginf(enc_g), 0.0, enc_g)
    enc = enc_g.reshape(N, T, H)
    parts_inp_obs = jnp.zeros((N, NUM_TIMEPTS, H), jnp.float32).at[:, :T].set(enc)
    parts_inp_msk = jnp.zeros((N, NUM_TIMEPTS, H), jnp.float32).at[:, :T].set(1.0)
    parts_inp_tps = jnp.zeros((N, NUM_TIMEPTS), jnp.float32).at[:, :T].set(
        pts_tid.astype(jnp.float32) / NUM_TIMEPTS)
    evd_obs = jnp.zeros((N, NUM_TIMEPTS, H), jnp.float32).at[
        jnp.arange(N)[:, None], pts_tid, :].set(enc)
    evd_msk = jnp.broadcast_to(pts_msk, (N, NUM_TIMEPTS, H))
    return (parts_inp_obs, parts_inp_msk, parts_inp_tps, evd_obs, evd_msk, pts_aux)


# speedup vs baseline: 1.8928x; 1.0077x over previous
import functools
import jax
import jax.numpy as jnp
from jax import lax
from jax.experimental import pallas as pl
from jax.experimental.pallas import tpu as pltpu
from jax.experimental.pallas import tpu_sc as plsc

N_NODES = 100000
N_EDGES = 1600000
NUM_GRAPHS = 200
N = 20
T = 10
NUM_TIMEPTS = 50
H = 32

NBLK = 2000
EBLK = 16384

NW = 32
GCHUNK = 128
MACRO = 512
CPW = 98
E_PAD = NW * MACRO * CPW


def _uv_block(x_ref, wu_ref, wv_ref, u_ref, v_ref):
    x = x_ref[...]
    u_ref[...] = jax.lax.dot_general(x, wu_ref[...], (((1,), (0,)), ((), ())),
                                     preferred_element_type=jnp.float32)
    v_ref[...] = jax.lax.dot_general(x, wv_ref[...], (((1,), (0,)), ((), ())),
                                     preferred_element_type=jnp.float32)


def _node_uv(x, Wu, Wv):
    F = x.shape[1]
    return pl.pallas_call(
        _uv_block,
        grid=(N_NODES // NBLK,),
        in_specs=[
            pl.BlockSpec((NBLK, F), lambda i: (i, 0)),
            pl.BlockSpec((F, H), lambda i: (0, 0)),
            pl.BlockSpec((F, H), lambda i: (0, 0)),
        ],
        out_specs=[pl.BlockSpec((NBLK, H), lambda i: (i, 0)),
                   pl.BlockSpec((NBLK, H), lambda i: (i, 0))],
        out_shape=[jax.ShapeDtypeStruct((N_NODES, H), jnp.float32),
                   jax.ShapeDtypeStruct((N_NODES, H), jnp.float32)],
    )(x, Wu, Wv)


def _gather_uv(u, v, src_pad, dst_pad):
    mesh = plsc.VectorSubcoreMesh(core_axis_name="c", subcore_axis_name="s")

    @functools.partial(
        pl.kernel, mesh=mesh,
        compiler_params=pltpu.CompilerParams(use_tc_tiling_on_sc=False),
        out_type=[jax.ShapeDtypeStruct((E_PAD, H), jnp.float32),
                  jax.ShapeDtypeStruct((E_PAD, H), jnp.float32)],
        scratch_types=[
            pltpu.VMEM((MACRO,), jnp.int32),
            pltpu.VMEM((MACRO,), jnp.int32),
            pltpu.VMEM((MACRO, H), jnp.float32),
            pltpu.VMEM((MACRO, H), jnp.float32),
            pltpu.VMEM((MACRO,), jnp.int32),
            pltpu.VMEM((MACRO,), jnp.int32),
            pltpu.VMEM((MACRO, H), jnp.float32),
            pltpu.VMEM((MACRO, H), jnp.float32),
            pltpu.SemaphoreType.DMA,
            pltpu.SemaphoreType.DMA,
        ],
    )
    def k(u_hbm, v_hbm, src_hbm, dst_hbm, gu_hbm, gv_hbm,
          idxs0, idxd0, bufu0, bufv0, idxs1, idxd1, bufu1, bufv1,
          sem0, sem1):
        wid = lax.axis_index("s") * 2 + lax.axis_index("c")
        base = wid * (MACRO * CPW)
        slots = ((idxs0, idxd0, bufu0, bufv0, sem0),
                 (idxs1, idxd1, bufu1, bufv1, sem1))

        def fire(c, s):
            ix, idd, bu, bv, sg = s
            off = base + c * MACRO
            pltpu.sync_copy(src_hbm.at[pl.ds(off, MACRO)], ix)
            pltpu.sync_copy(dst_hbm.at[pl.ds(off, MACRO)], idd)
            for g in range(MACRO // GCHUNK):
                sl = pl.ds(g * GCHUNK, GCHUNK)
                pltpu.async_copy(u_hbm.at[ix.at[sl]], bu.at[sl], sg)
                pltpu.async_copy(v_hbm.at[idd.at[sl]], bv.at[sl], sg)

        def drain_store(c, s):
            ix, idd, bu, bv, sg = s
            off = base + c * MACRO
            for g in range(MACRO // GCHUNK):
                sl = pl.ds(g * GCHUNK, GCHUNK)
                pltpu.make_async_copy(u_hbm.at[ix.at[sl]], bu.at[sl],
                                      sg).wait()
                pltpu.make_async_copy(v_hbm.at[idd.at[sl]], bv.at[sl],
                                      sg).wait()
            pltpu.sync_copy(bu, gu_hbm.at[pl.ds(off, MACRO)])
            pltpu.sync_copy(bv, gv_hbm.at[pl.ds(off, MACRO)])

        fire(0, slots[0])

        def body(i, carry):
            c0 = 2 * i
            fire(c0 + 1, slots[1])
            drain_store(c0, slots[0])

            @pl.when(c0 + 2 < CPW)
            def _():
                fire(c0 + 2, slots[0])

            drain_store(c0 + 1, slots[1])
            return carry

        lax.fori_loop(0, CPW // 2, body, 0)

    return k(u, v, src_pad, dst_pad)


def _msg_block(gu_ref, gv_ref, b1_ref, w2_ref, b2_ref, out_ref):
    t = jax.nn.relu(gu_ref[...] - gv_ref[...] + b1_ref[...])
    out_ref[...] = jax.lax.dot_general(
        t, w2_ref[...], (((1,), (0,)), ((), ())),
        preferred_element_type=jnp.float32) + b2_ref[...]


def _edge_msg(gu, gv, b1, W2, b2):
    E = gu.shape[0]
    return pl.pallas_call(
        _msg_block,
        grid=(E // EBLK,),
        in_specs=[
            pl.BlockSpec((EBLK, H), lambda i: (i, 0)),
            pl.BlockSpec((EBLK, H), lambda i: (i, 0)),
            pl.BlockSpec((1, H), lambda i: (0, 0)),
            pl.BlockSpec((H, H), lambda i: (0, 0)),
            pl.BlockSpec((1, H), lambda i: (0, 0)),
        ],
        out_specs=pl.BlockSpec((EBLK, H), lambda i: (i, 0)),
        out_shape=jax.ShapeDtypeStruct((E, H), jnp.float32),
    )(gu, gv, b1.reshape(1, H), W2, b2.reshape(1, H))


def _point_layer(h, pos, src_pad, dst_pad, dst_seg, W1, b1, W2, b2, hw):
    Wh, Wp = W1[:hw], W1[hw:]
    F = 8 if hw == 3 else 40
    if hw == 3:
        x = jnp.pad(pos, ((0, 0), (0, F - 3)))
        Wu = jnp.zeros((F, H), jnp.float32).at[:3].set(Wh + Wp)
        Wv = jnp.zeros((F, H), jnp.float32).at[:3].set(Wp)
    else:
        x = jnp.concatenate(
            [h, pos, jnp.zeros((N_NODES, F - hw - 3), jnp.float32)], axis=1)
        Wu = jnp.zeros((F, H), jnp.float32).at[:hw].set(Wh).at[hw:hw + 3].set(Wp)
        Wv = jnp.zeros((F, H), jnp.float32).at[hw:hw + 3].set(Wp)
    u, v = _node_uv(x, Wu, Wv)
    gu, gv = _gather_uv(u, v, src_pad, dst_pad)
    m = _edge_msg(gu, gv, b1, W2, b2)
    agg = jax.ops.segment_max(m, dst_seg, num_segments=N_NODES + 1)[:N_NODES]
    return jnp.where(jnp.isneginf(agg), 0.0, agg)


def kernel(pos, edge_index, batch, pts_tid, pts_msk, pts_aux,
           W1a, b1a, W2a, b2a, W1b, b1b, W2b, b2b):
    src, dst = edge_index[0], edge_index[1]
    pad = E_PAD - N_EDGES
    src_pad = jnp.concatenate([src, jnp.zeros((pad,), src.dtype)])
    dst_pad = jnp.concatenate([dst, jnp.zeros((pad,), dst.dtype)])
    dst_seg = jnp.concatenate(
        [dst, jnp.full((pad,), N_NODES, dst.dtype)])
    h = jax.nn.relu(_point_layer(
        pos, pos, src_pad, dst_pad, dst_seg, W1a, b1a, W2a, b2a, 3))
    h = jax.nn.relu(_point_layer(
        h, pos, src_pad, dst_pad, dst_seg, W1b, b1b, W2b, b2b, H))
    enc_g = jax.ops.segment_max(h, batch, num_segments=NUM_GRAPHS)
    enc_g = jnp.where(jnp.isneginf(enc_g), 0.0, enc_g)
    enc = enc_g.reshape(N, T, H)
    parts_inp_obs = jnp.zeros((N, NUM_TIMEPTS, H), jnp.float32).at[:, :T].set(enc)
    parts_inp_msk = jnp.zeros((N, NUM_TIMEPTS, H), jnp.float32).at[:, :T].set(1.0)
    parts_inp_tps = jnp.zeros((N, NUM_TIMEPTS), jnp.float32).at[:, :T].set(
        pts_tid.astype(jnp.float32) / NUM_TIMEPTS)
    evd_obs = jnp.zeros((N, NUM_TIMEPTS, H), jnp.float32).at[
        jnp.arange(N)[:, None], pts_tid, :].set(enc)
    evd_msk = jnp.broadcast_to(pts_msk, (N, NUM_TIMEPTS, H))
    return (parts_inp_obs, parts_inp_msk, parts_inp_tps, evd_obs, evd_msk, pts_aux)
